# Initial kernel scaffold; baseline (speedup 1.0000x reference)
import jax
import jax.numpy as jnp
from jax.experimental import pallas as pl


def kernel(x, edge_index, W1, b1, W2, b2, Wg, att_src, att_dst, bg, Wl1, bl1, Wl2, bl2, Wl3, bl3):
    raise NotImplementedError("write your pallas kernel here")


# speedup vs baseline: 24.4006x; 24.4006x over previous
import functools
import jax
import jax.numpy as jnp
from jax import lax
from jax.experimental import pallas as pl
from jax.experimental.pallas import tpu as pltpu
from jax.experimental.pallas import tpu_sc as plsc

N = 10000
E = 320000
NP = 10240
D = 64
NC = 2
NS = 16
NW = NC * NS
EPW = E // NW
CH = 128
NCH = (EPW + CH - 1) // CH
EPWP = NCH * CH
RPW = NP // NS

_f32 = jnp.float32


def _mesh():
    return plsc.VectorSubcoreMesh(core_axis_name="c", subcore_axis_name="s",
                                  num_cores=NC, num_subcores=NS)


def _zero_fill(zbuf):
    def body(i, _):
        for q in range(4):
            zbuf[i, pl.ds(q * 16, 16)] = jnp.zeros((16,), _f32)
        return 0
    lax.fori_loop(0, 128, body, 0)


def _deg_body(col_hbm, out_hbm, deg_sh, idxc_v, ones_v, zrow_v):
    c = lax.axis_index("c")
    s = lax.axis_index("s")
    w = c * NS + s
    pltpu.sync_copy(col_hbm.at[w], idxc_v)
    for q in range(8):
        ones_v[pl.ds(q * 16, 16)] = jnp.ones((16,), _f32)
        zrow_v[pl.ds(q * 16, 16)] = jnp.zeros((16,), _f32)
    for k in range(RPW // CH):
        pltpu.sync_copy(zrow_v, deg_sh.at[pl.ds(s * RPW + k * CH, CH)])
    plsc.subcore_barrier()

    def chunk(j, _):
        pltpu.sync_copy(ones_v, deg_sh.at[idxc_v.at[j]], add=True)
        return 0
    lax.fori_loop(0, NCH, chunk, 0)
    plsc.subcore_barrier()
    pltpu.sync_copy(deg_sh.at[pl.ds(s * RPW, RPW)],
                    out_hbm.at[c, pl.ds(s * RPW, RPW)])


def _deg_call(col_idx):
    f = pl.kernel(
        _deg_body,
        out_type=jax.ShapeDtypeStruct((NC, NP), _f32),
        mesh=_mesh(),
        scratch_types=[
            pltpu.VMEM_SHARED((NP,), _f32),
            pltpu.VMEM((NCH, CH), jnp.int32),
            pltpu.VMEM((CH,), _f32),
            pltpu.VMEM((CH,), _f32),
        ],
    )
    return f(col_idx)


def _sprop_body(u_hbm, row_hbm, col_hbm, out_hbm,
                acc_sh, idxr_v, idxc_v, gbuf):
    c = lax.axis_index("c")
    s = lax.axis_index("s")
    w = c * NS + s
    pltpu.sync_copy(row_hbm.at[w], idxr_v)
    pltpu.sync_copy(col_hbm.at[w], idxc_v)
    _zero_fill(gbuf)
    for k in range(RPW // CH):
        pltpu.sync_copy(gbuf, acc_sh.at[pl.ds(s * RPW + k * CH, CH)])
    plsc.subcore_barrier()

    def chunk(j, _):
        pltpu.sync_copy(u_hbm.at[idxr_v.at[j]], gbuf)
        pltpu.sync_copy(gbuf, acc_sh.at[idxc_v.at[j]], add=True)
        return 0
    lax.fori_loop(0, NCH, chunk, 0)
    plsc.subcore_barrier()
    pltpu.sync_copy(acc_sh.at[pl.ds(s * RPW, RPW)],
                    out_hbm.at[c, pl.ds(s * RPW, RPW)])


def _sprop_call(u, row_idx, col_idx):
    f = pl.kernel(
        _sprop_body,
        out_type=jax.ShapeDtypeStruct((NC, NP, D), _f32),
        mesh=_mesh(),
        compiler_params=pltpu.CompilerParams(use_tc_tiling_on_sc=False),
        scratch_types=[
            pltpu.VMEM_SHARED((NP, D), _f32),
            pltpu.VMEM((NCH, CH), jnp.int32),
            pltpu.VMEM((NCH, CH), jnp.int32),
            pltpu.VMEM((CH, D), _f32),
        ],
    )
    return f(u, row_idx, col_idx)


def _gat_body(g_hbm, asrc_hbm, adst_hbm, c_hbm, row_hbm, col_hbm,
              acc_out, den_out,
              acc_sh, den_sh,
              idxr_v, idxc_v, gbuf, eebuf, avbuf, bvbuf, c_v):
    c = lax.axis_index("c")
    s = lax.axis_index("s")
    w = c * NS + s
    pltpu.sync_copy(row_hbm.at[w], idxr_v)
    pltpu.sync_copy(col_hbm.at[w], idxc_v)
    pltpu.sync_copy(c_hbm.at[pl.ds(0, 16)], c_v)
    _zero_fill(gbuf)
    for q in range(8):
        eebuf[pl.ds(q * 16, 16)] = jnp.zeros((16,), _f32)
    for k in range(RPW // CH):
        pltpu.sync_copy(gbuf, acc_sh.at[pl.ds(s * RPW + k * CH, CH)])
        pltpu.sync_copy(eebuf, den_sh.at[pl.ds(s * RPW + k * CH, CH)])
    plsc.subcore_barrier()

    cv = c_v[...]

    def chunk(j, _):
        pltpu.sync_copy(asrc_hbm.at[idxr_v.at[j]], avbuf)
        pltpu.sync_copy(adst_hbm.at[idxc_v.at[j]], bvbuf)
        for i in range(CH // 16):
            sv = avbuf[pl.ds(i * 16, 16)] + bvbuf[pl.ds(i * 16, 16)]
            ev = jnp.where(sv > 0, sv, 0.2 * sv)
            eebuf[pl.ds(i * 16, 16)] = jnp.exp(ev - cv)
        pltpu.sync_copy(eebuf, den_sh.at[idxc_v.at[j]], add=True)
        pltpu.sync_copy(g_hbm.at[idxr_v.at[j]], gbuf)

        def scale(k, _):
            eev = eebuf[pl.ds(k * 16, 16)]
            for l in range(16):
                sc = jnp.full((16,), eev[l], _f32)
                r = k * 16 + l
                for q in range(4):
                    gbuf[r, pl.ds(q * 16, 16)] = gbuf[r, pl.ds(q * 16, 16)] * sc
            return 0
        lax.fori_loop(0, CH // 16, scale, 0)
        pltpu.sync_copy(gbuf, acc_sh.at[idxc_v.at[j]], add=True)
        return 0
    lax.fori_loop(0, NCH, chunk, 0)
    plsc.subcore_barrier()
    pltpu.sync_copy(acc_sh.at[pl.ds(s * RPW, RPW)],
                    acc_out.at[c, pl.ds(s * RPW, RPW)])
    pltpu.sync_copy(den_sh.at[pl.ds(s * RPW, RPW)],
                    den_out.at[c, pl.ds(s * RPW, RPW)])


def _gat_call(g, asrc, adst, c128, row_idx, col_idx):
    f = pl.kernel(
        _gat_body,
        out_type=(jax.ShapeDtypeStruct((NC, NP, D), _f32),
                  jax.ShapeDtypeStruct((NC, NP), _f32)),
        mesh=_mesh(),
        compiler_params=pltpu.CompilerParams(use_tc_tiling_on_sc=False),
        scratch_types=[
            pltpu.VMEM_SHARED((NP, D), _f32),
            pltpu.VMEM_SHARED((NP,), _f32),
            pltpu.VMEM((NCH, CH), jnp.int32),
            pltpu.VMEM((NCH, CH), jnp.int32),
            pltpu.VMEM((CH, D), _f32),
            pltpu.VMEM((CH,), _f32),
            pltpu.VMEM((CH,), _f32),
            pltpu.VMEM((CH,), _f32),
            pltpu.VMEM((16,), _f32),
        ],
    )
    return f(g, asrc, adst, c128, row_idx, col_idx)


def _mmT(a, w):
    return lax.dot_general(a, w, (((1,), (1,)), ((), ())),
                           preferred_element_type=_f32)


def _lrelu(v, s):
    return jnp.where(v > 0, v, s * v)


def _tc_a_body(degp_ref, xp_ref, w1_ref, dinv_ref, dinv2_ref, u1_ref):
    deg = degp_ref[0, :] + degp_ref[1, :] + 1.0
    dv = lax.rsqrt(deg)
    dinv_ref[0, :] = dv
    dinv2_ref[0, :] = dv * dv
    y = _mmT(xp_ref[...], w1_ref[...])
    u1_ref[...] = dv[:, None] * y


def _tc_glue_body(sp_ref, u_ref, dinv2_ref, out_ref):
    s = sp_ref[0] + sp_ref[1] + u_ref[...]
    out_ref[...] = dinv2_ref[0, :][:, None] * s


def _tc_conv_body(sp_ref, u_ref, dinv_ref, b_ref, w_ref, h_ref, un_ref):
    dv = dinv_ref[0, :]
    p = dv[:, None] * (sp_ref[0] + sp_ref[1] + u_ref[...]) + b_ref[0, :]
    h = _lrelu(p, 0.1)
    h_ref[...] = h
    un_ref[...] = dv[:, None] * _mmT(h, w_ref[...])


def _tc_gate_body(sp_ref, u_ref, dinv_ref, b_ref, wg_ref, asv_ref, adv_ref,
                  g_ref, asrc_ref, adst_ref, c_ref):
    dv = dinv_ref[0, :]
    p = dv[:, None] * (sp_ref[0] + sp_ref[1] + u_ref[...]) + b_ref[0, :]
    h2 = _lrelu(p, 0.1)
    g = _mmT(h2, wg_ref[...])
    g_ref[...] = g
    asrc = _mmT(asv_ref[...], g)
    adst = _mmT(adv_ref[...], g)
    asrc_ref[...] = asrc
    adst_ref[...] = adst
    msk = lax.broadcasted_iota(jnp.int32, (1, NP), 1) < N
    m = (jnp.max(jnp.where(msk, asrc, -1e30))
         + jnp.max(jnp.where(msk, adst, -1e30)))
    cc = jnp.where(m > 0, m, 0.2 * m)
    c_ref[...] = jnp.full((1, 128), cc, _f32)


def _tc_fin_body(accp_ref, denp_ref, asrc_ref, adst_ref, c_ref, g_ref,
                 bg_ref, h3_ref):
    cc = c_ref[0, 0]
    sv = asrc_ref[0, :] + adst_ref[0, :]
    es = jnp.exp(_lrelu(sv, 0.2) - cc)
    num = accp_ref[0] + accp_ref[1] + es[:, None] * g_ref[...]
    den = denp_ref[0, :] + denp_ref[1, :] + es
    h3_ref[...] = _lrelu(num / (den[:, None] + 1e-16) + bg_ref[0, :], 0.1)


def _tc_mlp_body(z_ref, w1_ref, b1_ref, w2_ref, b2_ref, w3_ref, b3_ref,
                 out_ref):
    o = _lrelu(_mmT(z_ref[...], w1_ref[...]) + b1_ref[0, :], 0.1)
    o = _lrelu(_mmT(o, w2_ref[...]) + b2_ref[0, :], 0.1)
    out_ref[...] = _lrelu(_mmT(o, w3_ref[...]) + b3_ref[0, :], 0.1)


def _tc_call(body, out_shapes, *args):
    return pl.pallas_call(
        body,
        out_shape=out_shapes,
    )(*args)


def kernel(x, edge_index, W1, b1, W2, b2, Wg, att_src, att_dst, bg,
           Wl1, bl1, Wl2, bl2, Wl3, bl3):
    row = edge_index[0]
    col = edge_index[1]
    pad = EPWP - EPW
    padrow = ((jnp.arange(NW, dtype=jnp.int32) * 317) % N)[:, None]
    row_idx = jnp.concatenate(
        [row.reshape(NW, EPW),
         jnp.broadcast_to(padrow, (NW, pad)).astype(jnp.int32)],
        axis=1).reshape(NW, NCH, CH)
    padcol = (N + (jnp.arange(NW, dtype=jnp.int32) % 128))[:, None]
    col_idx = jnp.concatenate(
        [col.reshape(NW, EPW),
         jnp.broadcast_to(padcol, (NW, pad)).astype(jnp.int32)],
        axis=1).reshape(NW, NCH, CH)

    xp = jnp.pad(x, ((0, NP - N), (0, 0)))

    degp = _deg_call(col_idx)

    dinv, dinv2, u1 = _tc_call(
        _tc_a_body,
        (jax.ShapeDtypeStruct((1, NP), _f32),
         jax.ShapeDtypeStruct((1, NP), _f32),
         jax.ShapeDtypeStruct((NP, D), _f32)),
        degp, xp, W1)

    s1 = _sprop_call(u1, row_idx, col_idx)
    u2 = _tc_call(_tc_glue_body, jax.ShapeDtypeStruct((NP, D), _f32),
                  s1, u1, dinv2)
    s2 = _sprop_call(u2, row_idx, col_idx)
    h1, u3 = _tc_call(
        _tc_conv_body,
        (jax.ShapeDtypeStruct((NP, D), _f32),
         jax.ShapeDtypeStruct((NP, D), _f32)),
        s2, u2, dinv, b1.reshape(1, D), W2)
    s3 = _sprop_call(u3, row_idx, col_idx)
    u4 = _tc_call(_tc_glue_body, jax.ShapeDtypeStruct((NP, D), _f32),
                  s3, u3, dinv2)
    s4 = _sprop_call(u4, row_idx, col_idx)

    g, asrc, adst, c128 = _tc_call(
        _tc_gate_body,
        (jax.ShapeDtypeStruct((NP, D), _f32),
         jax.ShapeDtypeStruct((1, NP), _f32),
         jax.ShapeDtypeStruct((1, NP), _f32),
         jax.ShapeDtypeStruct((1, 128), _f32)),
        s4, u4, dinv, b2.reshape(1, D), Wg,
        att_src.reshape(1, D), att_dst.reshape(1, D))

    accp, denp = _gat_call(g, asrc.reshape(NP), adst.reshape(NP),
                           c128.reshape(128), row_idx, col_idx)

    h3 = _tc_call(
        _tc_fin_body, jax.ShapeDtypeStruct((NP, D), _f32),
        accp, denp, asrc, adst, c128, g, bg.reshape(1, D))

    z = h3[:N].reshape(N // 40, 40 * D)
    out = _tc_call(
        _tc_mlp_body, jax.ShapeDtypeStruct((N // 40, 10), _f32),
        z, Wl1, bl1.reshape(1, -1), Wl2, bl2.reshape(1, -1),
        Wl3, bl3.reshape(1, -1))
    return out


# speedup vs baseline: 31.7332x; 1.3005x over previous
import functools
import jax
import jax.numpy as jnp
from jax import lax
from jax.experimental import pallas as pl
from jax.experimental.pallas import tpu as pltpu
from jax.experimental.pallas import tpu_sc as plsc

N = 10000
E = 320000
NP = 10240
D = 64
NC = 2
NS = 16
NW = NC * NS
EPW = E // NW
CH = 128
NCH = 80
EPWP = NCH * CH
RPW = NP // NS

_f32 = jnp.float32


def _mesh():
    return plsc.VectorSubcoreMesh(core_axis_name="c", subcore_axis_name="s",
                                  num_cores=NC, num_subcores=NS)


def _zero_fill(zbuf):
    def body(i, _):
        for q in range(4):
            zbuf[i, pl.ds(q * 16, 16)] = jnp.zeros((16,), _f32)
        return 0
    lax.fori_loop(0, 128, body, 0)


def _deg_body(col_hbm, out_hbm, deg_sh, idxc_v, ones_v, zrow_v):
    c = lax.axis_index("c")
    s = lax.axis_index("s")
    w = c * NS + s
    pltpu.sync_copy(col_hbm.at[w], idxc_v)
    for q in range(8):
        ones_v[pl.ds(q * 16, 16)] = jnp.ones((16,), _f32)
        zrow_v[pl.ds(q * 16, 16)] = jnp.zeros((16,), _f32)
    for k in range(RPW // CH):
        pltpu.sync_copy(zrow_v, deg_sh.at[pl.ds(s * RPW + k * CH, CH)])
    plsc.subcore_barrier()

    def chunk(j, _):
        pltpu.sync_copy(ones_v, deg_sh.at[idxc_v.at[j]], add=True)
        return 0
    lax.fori_loop(0, NCH, chunk, 0)
    plsc.subcore_barrier()
    pltpu.sync_copy(deg_sh.at[pl.ds(s * RPW, RPW)],
                    out_hbm.at[c, pl.ds(s * RPW, RPW)])


def _deg_call(col_idx):
    f = pl.kernel(
        _deg_body,
        out_type=jax.ShapeDtypeStruct((NC, NP), _f32),
        mesh=_mesh(),
        scratch_types=[
            pltpu.VMEM_SHARED((NP,), _f32),
            pltpu.VMEM((NCH, CH), jnp.int32),
            pltpu.VMEM((CH,), _f32),
            pltpu.VMEM((CH,), _f32),
        ],
    )
    return f(col_idx)


def _sprop_body(u_hbm, row_hbm, col_hbm, out_hbm,
                acc_sh, idxr_v, idxc_v, g0, g1, semA, semB):
    c = lax.axis_index("c")
    s = lax.axis_index("s")
    w = c * NS + s
    pltpu.sync_copy(row_hbm.at[w], idxr_v)
    pltpu.sync_copy(col_hbm.at[w], idxc_v)
    _zero_fill(g0)
    for k in range(RPW // CH):
        pltpu.sync_copy(g0, acc_sh.at[pl.ds(s * RPW + k * CH, CH)])
    plsc.subcore_barrier()

    pltpu.async_copy(u_hbm.at[idxr_v.at[0]], g0, semA)

    def dbl(t, _):
        j0 = 2 * t
        j1 = j0 + 1
        pltpu.make_async_copy(u_hbm.at[idxr_v.at[j0]], g0, semA).wait()
        pltpu.async_copy(u_hbm.at[idxr_v.at[j1]], g1, semB)
        pltpu.sync_copy(g0, acc_sh.at[idxc_v.at[j0]], add=True)
        pltpu.make_async_copy(u_hbm.at[idxr_v.at[j1]], g1, semB).wait()

        @pl.when(j0 + 2 < NCH)
        def _():
            pltpu.async_copy(u_hbm.at[idxr_v.at[j0 + 2]], g0, semA)
        pltpu.sync_copy(g1, acc_sh.at[idxc_v.at[j1]], add=True)
        return 0
    lax.fori_loop(0, NCH // 2, dbl, 0)
    plsc.subcore_barrier()
    pltpu.sync_copy(acc_sh.at[pl.ds(s * RPW, RPW)],
                    out_hbm.at[c, pl.ds(s * RPW, RPW)])


def _sprop_call(u, row_idx, col_idx):
    f = pl.kernel(
        _sprop_body,
        out_type=jax.ShapeDtypeStruct((NC, NP, D), _f32),
        mesh=_mesh(),
        compiler_params=pltpu.CompilerParams(use_tc_tiling_on_sc=False),
        scratch_types=[
            pltpu.VMEM_SHARED((NP, D), _f32),
            pltpu.VMEM((NCH, CH), jnp.int32),
            pltpu.VMEM((NCH, CH), jnp.int32),
            pltpu.VMEM((CH, D), _f32),
            pltpu.VMEM((CH, D), _f32),
            pltpu.SemaphoreType.DMA,
            pltpu.SemaphoreType.DMA,
        ],
    )
    return f(u, row_idx, col_idx)


def _gat_body(g_hbm, asrc_hbm, adst_hbm, c_hbm, row_hbm, col_hbm,
              acc_out, den_out,
              acc_sh, den_sh,
              idxr_v, idxc_v, g0, g1, av0, av1, bv0, bv1, eebuf, c_v,
              sg0, sg1, sa0, sa1, sb0, sb1):
    c = lax.axis_index("c")
    s = lax.axis_index("s")
    w = c * NS + s
    pltpu.sync_copy(row_hbm.at[w], idxr_v)
    pltpu.sync_copy(col_hbm.at[w], idxc_v)
    pltpu.sync_copy(c_hbm.at[pl.ds(0, 16)], c_v)
    _zero_fill(g0)
    for q in range(8):
        eebuf[pl.ds(q * 16, 16)] = jnp.zeros((16,), _f32)
    for k in range(RPW // CH):
        pltpu.sync_copy(g0, acc_sh.at[pl.ds(s * RPW + k * CH, CH)])
        pltpu.sync_copy(eebuf, den_sh.at[pl.ds(s * RPW + k * CH, CH)])
    plsc.subcore_barrier()

    cv = c_v[...]

    pltpu.async_copy(asrc_hbm.at[idxr_v.at[0]], av0, sa0)
    pltpu.async_copy(adst_hbm.at[idxc_v.at[0]], bv0, sb0)
    pltpu.async_copy(g_hbm.at[idxr_v.at[0]], g0, sg0)

    def half(j, jn, ga, sga, ava, sava, bva, sbva, gb, sgb, avb, savb, bvb,
             sbvb):
        @pl.when(jn < NCH)
        def _():
            pltpu.async_copy(asrc_hbm.at[idxr_v.at[jn]], avb, savb)
            pltpu.async_copy(adst_hbm.at[idxc_v.at[jn]], bvb, sbvb)
            pltpu.async_copy(g_hbm.at[idxr_v.at[jn]], gb, sgb)
        pltpu.make_async_copy(asrc_hbm.at[idxr_v.at[j]], ava, sava).wait()
        pltpu.make_async_copy(adst_hbm.at[idxc_v.at[j]], bva, sbva).wait()
        for i in range(CH // 16):
            sv = ava[pl.ds(i * 16, 16)] + bva[pl.ds(i * 16, 16)]
            ev = jnp.where(sv > 0, sv, 0.2 * sv)
            eebuf[pl.ds(i * 16, 16)] = jnp.exp(ev - cv)
        pltpu.sync_copy(eebuf, den_sh.at[idxc_v.at[j]], add=True)
        pltpu.make_async_copy(g_hbm.at[idxr_v.at[j]], ga, sga).wait()

        def scale(k, _):
            eev = eebuf[pl.ds(k * 16, 16)]
            for l in range(16):
                sc = jnp.full((16,), eev[l], _f32)
                r = k * 16 + l
                for q in range(4):
                    ga[r, pl.ds(q * 16, 16)] = ga[r, pl.ds(q * 16, 16)] * sc
            return 0
        lax.fori_loop(0, CH // 16, scale, 0)
        pltpu.sync_copy(ga, acc_sh.at[idxc_v.at[j]], add=True)

    def dbl(t, _):
        j0 = 2 * t
        j1 = j0 + 1
        half(j0, j1, g0, sg0, av0, sa0, bv0, sb0,
             g1, sg1, av1, sa1, bv1, sb1)
        half(j1, j0 + 2, g1, sg1, av1, sa1, bv1, sb1,
             g0, sg0, av0, sa0, bv0, sb0)
        return 0
    lax.fori_loop(0, NCH // 2, dbl, 0)
    plsc.subcore_barrier()
    pltpu.sync_copy(acc_sh.at[pl.ds(s * RPW, RPW)],
                    acc_out.at[c, pl.ds(s * RPW, RPW)])
    pltpu.sync_copy(den_sh.at[pl.ds(s * RPW, RPW)],
                    den_out.at[c, pl.ds(s * RPW, RPW)])


def _gat_call(g, asrc, adst, c128, row_idx, col_idx):
    f = pl.kernel(
        _gat_body,
        out_type=(jax.ShapeDtypeStruct((NC, NP, D), _f32),
                  jax.ShapeDtypeStruct((NC, NP), _f32)),
        mesh=_mesh(),
        compiler_params=pltpu.CompilerParams(use_tc_tiling_on_sc=False),
        scratch_types=[
            pltpu.VMEM_SHARED((NP, D), _f32),
            pltpu.VMEM_SHARED((NP,), _f32),
            pltpu.VMEM((NCH, CH), jnp.int32),
            pltpu.VMEM((NCH, CH), jnp.int32),
            pltpu.VMEM((CH, D), _f32),
            pltpu.VMEM((CH, D), _f32),
            pltpu.VMEM((CH,), _f32),
            pltpu.VMEM((CH,), _f32),
            pltpu.VMEM((CH,), _f32),
            pltpu.VMEM((CH,), _f32),
            pltpu.VMEM((CH,), _f32),
            pltpu.VMEM((16,), _f32),
            pltpu.SemaphoreType.DMA,
            pltpu.SemaphoreType.DMA,
            pltpu.SemaphoreType.DMA,
            pltpu.SemaphoreType.DMA,
            pltpu.SemaphoreType.DMA,
            pltpu.SemaphoreType.DMA,
        ],
    )
    return f(g, asrc, adst, c128, row_idx, col_idx)


def _mmT(a, w):
    return lax.dot_general(a, w, (((1,), (1,)), ((), ())),
                           preferred_element_type=_f32)


def _lrelu(v, s):
    return jnp.where(v > 0, v, s * v)


def _tc_a_body(degp_ref, xp_ref, w1_ref, dinv_ref, dinv2_ref, u1_ref):
    deg = degp_ref[0, :] + degp_ref[1, :] + 1.0
    dv = lax.rsqrt(deg)
    dinv_ref[0, :] = dv
    dinv2_ref[0, :] = dv * dv
    y = _mmT(xp_ref[...], w1_ref[...])
    u1_ref[...] = dv[:, None] * y


def _tc_glue_body(sp_ref, u_ref, dinv2_ref, out_ref):
    s = sp_ref[0] + sp_ref[1] + u_ref[...]
    out_ref[...] = dinv2_ref[0, :][:, None] * s


def _tc_conv_body(sp_ref, u_ref, dinv_ref, b_ref, w_ref, h_ref, un_ref):
    dv = dinv_ref[0, :]
    p = dv[:, None] * (sp_ref[0] + sp_ref[1] + u_ref[...]) + b_ref[0, :]
    h = _lrelu(p, 0.1)
    h_ref[...] = h
    un_ref[...] = dv[:, None] * _mmT(h, w_ref[...])


def _tc_gate_body(sp_ref, u_ref, dinv_ref, b_ref, wg_ref, asv_ref, adv_ref,
                  g_ref, asrc_ref, adst_ref, c_ref):
    dv = dinv_ref[0, :]
    p = dv[:, None] * (sp_ref[0] + sp_ref[1] + u_ref[...]) + b_ref[0, :]
    h2 = _lrelu(p, 0.1)
    g = _mmT(h2, wg_ref[...])
    g_ref[...] = g
    asrc = _mmT(asv_ref[...], g)
    adst = _mmT(adv_ref[...], g)
    asrc_ref[...] = asrc
    adst_ref[...] = adst
    msk = lax.broadcasted_iota(jnp.int32, (1, NP), 1) < N
    m = (jnp.max(jnp.where(msk, asrc, -1e30))
         + jnp.max(jnp.where(msk, adst, -1e30)))
    cc = jnp.where(m > 0, m, 0.2 * m)
    c_ref[...] = jnp.full((1, 128), cc, _f32)


def _tc_fin_body(accp_ref, denp_ref, asrc_ref, adst_ref, c_ref, g_ref,
                 bg_ref, h3_ref):
    cc = c_ref[0, 0]
    sv = asrc_ref[0, :] + adst_ref[0, :]
    es = jnp.exp(_lrelu(sv, 0.2) - cc)
    num = accp_ref[0] + accp_ref[1] + es[:, None] * g_ref[...]
    den = denp_ref[0, :] + denp_ref[1, :] + es
    h3_ref[...] = _lrelu(num / (den[:, None] + 1e-16) + bg_ref[0, :], 0.1)


def _tc_mlp_body(z_ref, w1_ref, b1_ref, w2_ref, b2_ref, w3_ref, b3_ref,
                 out_ref):
    o = _lrelu(_mmT(z_ref[...], w1_ref[...]) + b1_ref[0, :], 0.1)
    o = _lrelu(_mmT(o, w2_ref[...]) + b2_ref[0, :], 0.1)
    out_ref[...] = _lrelu(_mmT(o, w3_ref[...]) + b3_ref[0, :], 0.1)


def _tc_call(body, out_shapes, *args):
    return pl.pallas_call(
        body,
        out_shape=out_shapes,
    )(*args)


def kernel(x, edge_index, W1, b1, W2, b2, Wg, att_src, att_dst, bg,
           Wl1, bl1, Wl2, bl2, Wl3, bl3):
    row = edge_index[0]
    col = edge_index[1]
    pad = EPWP - EPW
    padrow = ((jnp.arange(NW, dtype=jnp.int32) * 317) % N)[:, None]
    row_idx = jnp.concatenate(
        [row.reshape(NW, EPW),
         jnp.broadcast_to(padrow, (NW, pad)).astype(jnp.int32)],
        axis=1).reshape(NW, NCH, CH)
    padcol = (N + (jnp.arange(NW, dtype=jnp.int32) % 128))[:, None]
    col_idx = jnp.concatenate(
        [col.reshape(NW, EPW),
         jnp.broadcast_to(padcol, (NW, pad)).astype(jnp.int32)],
        axis=1).reshape(NW, NCH, CH)

    xp = jnp.pad(x, ((0, NP - N), (0, 0)))

    degp = _deg_call(col_idx)

    dinv, dinv2, u1 = _tc_call(
        _tc_a_body,
        (jax.ShapeDtypeStruct((1, NP), _f32),
         jax.ShapeDtypeStruct((1, NP), _f32),
         jax.ShapeDtypeStruct((NP, D), _f32)),
        degp, xp, W1)

    s1 = _sprop_call(u1, row_idx, col_idx)
    u2 = _tc_call(_tc_glue_body, jax.ShapeDtypeStruct((NP, D), _f32),
                  s1, u1, dinv2)
    s2 = _sprop_call(u2, row_idx, col_idx)
    h1, u3 = _tc_call(
        _tc_conv_body,
        (jax.ShapeDtypeStruct((NP, D), _f32),
         jax.ShapeDtypeStruct((NP, D), _f32)),
        s2, u2, dinv, b1.reshape(1, D), W2)
    s3 = _sprop_call(u3, row_idx, col_idx)
    u4 = _tc_call(_tc_glue_body, jax.ShapeDtypeStruct((NP, D), _f32),
                  s3, u3, dinv2)
    s4 = _sprop_call(u4, row_idx, col_idx)

    g, asrc, adst, c128 = _tc_call(
        _tc_gate_body,
        (jax.ShapeDtypeStruct((NP, D), _f32),
         jax.ShapeDtypeStruct((1, NP), _f32),
         jax.ShapeDtypeStruct((1, NP), _f32),
         jax.ShapeDtypeStruct((1, 128), _f32)),
        s4, u4, dinv, b2.reshape(1, D), Wg,
        att_src.reshape(1, D), att_dst.reshape(1, D))

    accp, denp = _gat_call(g, asrc.reshape(NP), adst.reshape(NP),
                           c128.reshape(128), row_idx, col_idx)

    h3 = _tc_call(
        _tc_fin_body, jax.ShapeDtypeStruct((NP, D), _f32),
        accp, denp, asrc, adst, c128, g, bg.reshape(1, D))

    z = h3[:N].reshape(N // 40, 40 * D)
    out = _tc_call(
        _tc_mlp_body, jax.ShapeDtypeStruct((N // 40, 10), _f32),
        z, Wl1, bl1.reshape(1, -1), Wl2, bl2.reshape(1, -1),
        Wl3, bl3.reshape(1, -1))
    return out


# speedup vs baseline: 36.2850x; 1.1434x over previous
import functools
import jax
import jax.numpy as jnp
from jax import lax
from jax.experimental import pallas as pl
from jax.experimental.pallas import tpu as pltpu
from jax.experimental.pallas import tpu_sc as plsc

N = 10000
E = 320000
NP = 10240
D = 64
NC = 2
NS = 16
NW = NC * NS
EPW = E // NW
CH = 128
NCH = 80
EPWP = NCH * CH
RPW = NP // NS

_f32 = jnp.float32


def _mesh():
    return plsc.VectorSubcoreMesh(core_axis_name="c", subcore_axis_name="s",
                                  num_cores=NC, num_subcores=NS)


def _zero_fill(zbuf):
    def body(i, _):
        for q in range(4):
            zbuf[i, pl.ds(q * 16, 16)] = jnp.zeros((16,), _f32)
        return 0
    lax.fori_loop(0, 128, body, 0)


def _deg_body(col_hbm, out_hbm, deg_sh, idxc_v, ones_v, zrow_v):
    c = lax.axis_index("c")
    s = lax.axis_index("s")
    w = c * NS + s
    pltpu.sync_copy(col_hbm.at[w], idxc_v)
    for q in range(8):
        ones_v[pl.ds(q * 16, 16)] = jnp.ones((16,), _f32)
        zrow_v[pl.ds(q * 16, 16)] = jnp.zeros((16,), _f32)
    for k in range(RPW // CH):
        pltpu.sync_copy(zrow_v, deg_sh.at[pl.ds(s * RPW + k * CH, CH)])
    plsc.subcore_barrier()

    def chunk(j, _):
        pltpu.sync_copy(ones_v, deg_sh.at[idxc_v.at[j]], add=True)
        return 0
    lax.fori_loop(0, NCH, chunk, 0)
    plsc.subcore_barrier()
    pltpu.sync_copy(deg_sh.at[pl.ds(s * RPW, RPW)],
                    out_hbm.at[c, pl.ds(s * RPW, RPW)])


def _deg_call(col_idx):
    f = pl.kernel(
        _deg_body,
        out_type=jax.ShapeDtypeStruct((NC, NP), _f32),
        mesh=_mesh(),
        scratch_types=[
            pltpu.VMEM_SHARED((NP,), _f32),
            pltpu.VMEM((NCH, CH), jnp.int32),
            pltpu.VMEM((CH,), _f32),
            pltpu.VMEM((CH,), _f32),
        ],
    )
    return f(col_idx)


def _sprop_body(u_hbm, row_hbm, col_hbm, out_hbm,
                acc_sh, idxr_v, idxc_v, b0, b1, b2, b3,
                sg0, sg1, sg2, sg3, ss0, ss1, ss2, ss3):
    c = lax.axis_index("c")
    s = lax.axis_index("s")
    w = c * NS + s
    bufs = (b0, b1, b2, b3)
    sgs = (sg0, sg1, sg2, sg3)
    sss = (ss0, ss1, ss2, ss3)
    pltpu.sync_copy(row_hbm.at[w], idxr_v)
    pltpu.sync_copy(col_hbm.at[w], idxc_v)
    _zero_fill(b0)
    for k in range(RPW // CH):
        pltpu.sync_copy(b0, acc_sh.at[pl.ds(s * RPW + k * CH, CH)])
    plsc.subcore_barrier()

    pltpu.async_copy(u_hbm.at[idxr_v.at[0]], b0, sg0)
    pltpu.async_copy(u_hbm.at[idxr_v.at[1]], b1, sg1)

    def quad(t, _):
        for u in range(4):
            j = 4 * t + u
            v = (u + 2) % 4
            pltpu.make_async_copy(u_hbm.at[idxr_v.at[j]], bufs[u],
                                  sgs[u]).wait()
            pltpu.async_copy(bufs[u], acc_sh.at[idxc_v.at[j]], sss[u],
                             add=True)

            @pl.when(j >= 2)
            def _():
                pltpu.make_async_copy(bufs[v], acc_sh.at[idxc_v.at[j - 2]],
                                      sss[v]).wait()

            @pl.when(j + 2 < NCH)
            def _():
                pltpu.async_copy(u_hbm.at[idxr_v.at[j + 2]], bufs[v], sgs[v])
        return 0
    lax.fori_loop(0, NCH // 4, quad, 0)
    pltpu.make_async_copy(b2, acc_sh.at[idxc_v.at[NCH - 2]], ss2).wait()
    pltpu.make_async_copy(b3, acc_sh.at[idxc_v.at(NCH - 1) if False else idxc_v.at[NCH - 1]], ss3).wait()
    plsc.subcore_barrier()
    pltpu.sync_copy(acc_sh.at[pl.ds(s * RPW, RPW)],
                    out_hbm.at[c, pl.ds(s * RPW, RPW)])


def _sprop_call(u, row_idx, col_idx):
    f = pl.kernel(
        _sprop_body,
        out_type=jax.ShapeDtypeStruct((NC, NP, D), _f32),
        mesh=_mesh(),
        compiler_params=pltpu.CompilerParams(use_tc_tiling_on_sc=False),
        scratch_types=[
            pltpu.VMEM_SHARED((NP, D), _f32),
            pltpu.VMEM((NCH, CH), jnp.int32),
            pltpu.VMEM((NCH, CH), jnp.int32),
            pltpu.VMEM((CH, D), _f32),
            pltpu.VMEM((CH, D), _f32),
            pltpu.VMEM((CH, D), _f32),
            pltpu.VMEM((CH, D), _f32),
            pltpu.SemaphoreType.DMA,
            pltpu.SemaphoreType.DMA,
            pltpu.SemaphoreType.DMA,
            pltpu.SemaphoreType.DMA,
            pltpu.SemaphoreType.DMA,
            pltpu.SemaphoreType.DMA,
            pltpu.SemaphoreType.DMA,
            pltpu.SemaphoreType.DMA,
        ],
    )
    return f(u, row_idx, col_idx)


def _gat_body(g_hbm, asrc_hbm, adst_hbm, c_hbm, row_hbm, col_hbm,
              acc_out, den_out,
              acc_sh, den_sh,
              idxr_v, idxc_v, g0, g1, av0, av1, bv0, bv1, eebuf, c_v,
              sg0, sg1, sa0, sa1, sb0, sb1):
    c = lax.axis_index("c")
    s = lax.axis_index("s")
    w = c * NS + s
    pltpu.sync_copy(row_hbm.at[w], idxr_v)
    pltpu.sync_copy(col_hbm.at[w], idxc_v)
    pltpu.sync_copy(c_hbm.at[pl.ds(0, 16)], c_v)
    _zero_fill(g0)
    for q in range(8):
        eebuf[pl.ds(q * 16, 16)] = jnp.zeros((16,), _f32)
    for k in range(RPW // CH):
        pltpu.sync_copy(g0, acc_sh.at[pl.ds(s * RPW + k * CH, CH)])
        pltpu.sync_copy(eebuf, den_sh.at[pl.ds(s * RPW + k * CH, CH)])
    plsc.subcore_barrier()

    cv = c_v[...]

    pltpu.async_copy(asrc_hbm.at[idxr_v.at[0]], av0, sa0)
    pltpu.async_copy(adst_hbm.at[idxc_v.at[0]], bv0, sb0)
    pltpu.async_copy(g_hbm.at[idxr_v.at[0]], g0, sg0)

    def half(j, jn, ga, sga, ava, sava, bva, sbva, gb, sgb, avb, savb, bvb,
             sbvb):
        @pl.when(jn < NCH)
        def _():
            pltpu.async_copy(asrc_hbm.at[idxr_v.at[jn]], avb, savb)
            pltpu.async_copy(adst_hbm.at[idxc_v.at[jn]], bvb, sbvb)
            pltpu.async_copy(g_hbm.at[idxr_v.at[jn]], gb, sgb)
        pltpu.make_async_copy(asrc_hbm.at[idxr_v.at[j]], ava, sava).wait()
        pltpu.make_async_copy(adst_hbm.at[idxc_v.at[j]], bva, sbva).wait()
        for i in range(CH // 16):
            sv = ava[pl.ds(i * 16, 16)] + bva[pl.ds(i * 16, 16)]
            ev = jnp.where(sv > 0, sv, 0.2 * sv)
            eebuf[pl.ds(i * 16, 16)] = jnp.exp(ev - cv)
        pltpu.sync_copy(eebuf, den_sh.at[idxc_v.at[j]], add=True)
        pltpu.make_async_copy(g_hbm.at[idxr_v.at[j]], ga, sga).wait()

        def scale(k, _):
            eev = eebuf[pl.ds(k * 16, 16)]
            for l in range(16):
                sc = jnp.full((16,), eev[l], _f32)
                r = k * 16 + l
                for q in range(4):
                    ga[r, pl.ds(q * 16, 16)] = ga[r, pl.ds(q * 16, 16)] * sc
            return 0
        lax.fori_loop(0, CH // 16, scale, 0)
        pltpu.sync_copy(ga, acc_sh.at[idxc_v.at[j]], add=True)

    def dbl(t, _):
        j0 = 2 * t
        j1 = j0 + 1
        half(j0, j1, g0, sg0, av0, sa0, bv0, sb0,
             g1, sg1, av1, sa1, bv1, sb1)
        half(j1, j0 + 2, g1, sg1, av1, sa1, bv1, sb1,
             g0, sg0, av0, sa0, bv0, sb0)
        return 0
    lax.fori_loop(0, NCH // 2, dbl, 0)
    plsc.subcore_barrier()
    pltpu.sync_copy(acc_sh.at[pl.ds(s * RPW, RPW)],
                    acc_out.at[c, pl.ds(s * RPW, RPW)])
    pltpu.sync_copy(den_sh.at[pl.ds(s * RPW, RPW)],
                    den_out.at[c, pl.ds(s * RPW, RPW)])


def _gat_call(g, asrc, adst, c128, row_idx, col_idx):
    f = pl.kernel(
        _gat_body,
        out_type=(jax.ShapeDtypeStruct((NC, NP, D), _f32),
                  jax.ShapeDtypeStruct((NC, NP), _f32)),
        mesh=_mesh(),
        compiler_params=pltpu.CompilerParams(use_tc_tiling_on_sc=False),
        scratch_types=[
            pltpu.VMEM_SHARED((NP, D), _f32),
            pltpu.VMEM_SHARED((NP,), _f32),
            pltpu.VMEM((NCH, CH), jnp.int32),
            pltpu.VMEM((NCH, CH), jnp.int32),
            pltpu.VMEM((CH, D), _f32),
            pltpu.VMEM((CH, D), _f32),
            pltpu.VMEM((CH,), _f32),
            pltpu.VMEM((CH,), _f32),
            pltpu.VMEM((CH,), _f32),
            pltpu.VMEM((CH,), _f32),
            pltpu.VMEM((CH,), _f32),
            pltpu.VMEM((16,), _f32),
            pltpu.SemaphoreType.DMA,
            pltpu.SemaphoreType.DMA,
            pltpu.SemaphoreType.DMA,
            pltpu.SemaphoreType.DMA,
            pltpu.SemaphoreType.DMA,
            pltpu.SemaphoreType.DMA,
        ],
    )
    return f(g, asrc, adst, c128, row_idx, col_idx)


def _mmT(a, w):
    return lax.dot_general(a, w, (((1,), (1,)), ((), ())),
                           preferred_element_type=_f32)


def _lrelu(v, s):
    return jnp.where(v > 0, v, s * v)


def _tc_a_body(degp_ref, xp_ref, w1_ref, dinv_ref, dinv2_ref, u1_ref):
    deg = degp_ref[0, :] + degp_ref[1, :] + 1.0
    dv = lax.rsqrt(deg)
    dinv_ref[0, :] = dv
    dinv2_ref[0, :] = dv * dv
    y = _mmT(xp_ref[...], w1_ref[...])
    u1_ref[...] = dv[:, None] * y


def _tc_glue_body(sp_ref, u_ref, dinv2_ref, out_ref):
    s = sp_ref[0] + sp_ref[1] + u_ref[...]
    out_ref[...] = dinv2_ref[0, :][:, None] * s


def _tc_conv_body(sp_ref, u_ref, dinv_ref, b_ref, w_ref, h_ref, un_ref):
    dv = dinv_ref[0, :]
    p = dv[:, None] * (sp_ref[0] + sp_ref[1] + u_ref[...]) + b_ref[0, :]
    h = _lrelu(p, 0.1)
    h_ref[...] = h
    un_ref[...] = dv[:, None] * _mmT(h, w_ref[...])


def _tc_gate_body(sp_ref, u_ref, dinv_ref, b_ref, wg_ref, asv_ref, adv_ref,
                  g_ref, asrc_ref, adst_ref, c_ref):
    dv = dinv_ref[0, :]
    p = dv[:, None] * (sp_ref[0] + sp_ref[1] + u_ref[...]) + b_ref[0, :]
    h2 = _lrelu(p, 0.1)
    g = _mmT(h2, wg_ref[...])
    g_ref[...] = g
    asrc = _mmT(asv_ref[...], g)
    adst = _mmT(adv_ref[...], g)
    asrc_ref[...] = asrc
    adst_ref[...] = adst
    msk = lax.broadcasted_iota(jnp.int32, (1, NP), 1) < N
    m = (jnp.max(jnp.where(msk, asrc, -1e30))
         + jnp.max(jnp.where(msk, adst, -1e30)))
    cc = jnp.where(m > 0, m, 0.2 * m)
    c_ref[...] = jnp.full((1, 128), cc, _f32)


def _tc_fin_body(accp_ref, denp_ref, asrc_ref, adst_ref, c_ref, g_ref,
                 bg_ref, h3_ref):
    cc = c_ref[0, 0]
    sv = asrc_ref[0, :] + adst_ref[0, :]
    es = jnp.exp(_lrelu(sv, 0.2) - cc)
    num = accp_ref[0] + accp_ref[1] + es[:, None] * g_ref[...]
    den = denp_ref[0, :] + denp_ref[1, :] + es
    h3_ref[...] = _lrelu(num / (den[:, None] + 1e-16) + bg_ref[0, :], 0.1)


def _tc_mlp_body(z_ref, w1_ref, b1_ref, w2_ref, b2_ref, w3_ref, b3_ref,
                 out_ref):
    o = _lrelu(_mmT(z_ref[...], w1_ref[...]) + b1_ref[0, :], 0.1)
    o = _lrelu(_mmT(o, w2_ref[...]) + b2_ref[0, :], 0.1)
    out_ref[...] = _lrelu(_mmT(o, w3_ref[...]) + b3_ref[0, :], 0.1)


def _tc_call(body, out_shapes, *args):
    return pl.pallas_call(
        body,
        out_shape=out_shapes,
    )(*args)


def kernel(x, edge_index, W1, b1, W2, b2, Wg, att_src, att_dst, bg,
           Wl1, bl1, Wl2, bl2, Wl3, bl3):
    row = edge_index[0]
    col = edge_index[1]
    pad = EPWP - EPW
    padrow = ((jnp.arange(NW, dtype=jnp.int32) * 317) % N)[:, None]
    row_idx = jnp.concatenate(
        [row.reshape(NW, EPW),
         jnp.broadcast_to(padrow, (NW, pad)).astype(jnp.int32)],
        axis=1).reshape(NW, NCH, CH)
    padcol = (N + (jnp.arange(NW, dtype=jnp.int32) % 128))[:, None]
    col_idx = jnp.concatenate(
        [col.reshape(NW, EPW),
         jnp.broadcast_to(padcol, (NW, pad)).astype(jnp.int32)],
        axis=1).reshape(NW, NCH, CH)

    xp = jnp.pad(x, ((0, NP - N), (0, 0)))

    degp = _deg_call(col_idx)

    dinv, dinv2, u1 = _tc_call(
        _tc_a_body,
        (jax.ShapeDtypeStruct((1, NP), _f32),
         jax.ShapeDtypeStruct((1, NP), _f32),
         jax.ShapeDtypeStruct((NP, D), _f32)),
        degp, xp, W1)

    s1 = _sprop_call(u1, row_idx, col_idx)
    u2 = _tc_call(_tc_glue_body, jax.ShapeDtypeStruct((NP, D), _f32),
                  s1, u1, dinv2)
    s2 = _sprop_call(u2, row_idx, col_idx)
    h1, u3 = _tc_call(
        _tc_conv_body,
        (jax.ShapeDtypeStruct((NP, D), _f32),
         jax.ShapeDtypeStruct((NP, D), _f32)),
        s2, u2, dinv, b1.reshape(1, D), W2)
    s3 = _sprop_call(u3, row_idx, col_idx)
    u4 = _tc_call(_tc_glue_body, jax.ShapeDtypeStruct((NP, D), _f32),
                  s3, u3, dinv2)
    s4 = _sprop_call(u4, row_idx, col_idx)

    g, asrc, adst, c128 = _tc_call(
        _tc_gate_body,
        (jax.ShapeDtypeStruct((NP, D), _f32),
         jax.ShapeDtypeStruct((1, NP), _f32),
         jax.ShapeDtypeStruct((1, NP), _f32),
         jax.ShapeDtypeStruct((1, 128), _f32)),
        s4, u4, dinv, b2.reshape(1, D), Wg,
        att_src.reshape(1, D), att_dst.reshape(1, D))

    accp, denp = _gat_call(g, asrc.reshape(NP), adst.reshape(NP),
                           c128.reshape(128), row_idx, col_idx)

    h3 = _tc_call(
        _tc_fin_body, jax.ShapeDtypeStruct((NP, D), _f32),
        accp, denp, asrc, adst, c128, g, bg.reshape(1, D))

    z = h3[:N].reshape(N // 40, 40 * D)
    out = _tc_call(
        _tc_mlp_body, jax.ShapeDtypeStruct((N // 40, 10), _f32),
        z, Wl1, bl1.reshape(1, -1), Wl2, bl2.reshape(1, -1),
        Wl3, bl3.reshape(1, -1))
    return out


# speedup vs baseline: 38.1271x; 1.0508x over previous
import functools
import jax
import jax.numpy as jnp
from jax import lax
from jax.experimental import pallas as pl
from jax.experimental.pallas import tpu as pltpu
from jax.experimental.pallas import tpu_sc as plsc

N = 10000
E = 320000
NP = 10240
D = 64
NC = 2
NS = 16
NW = NC * NS
EPW = E // NW
CH = 128
NCH = 80
EPWP = NCH * CH
RPW = NP // NS

_f32 = jnp.float32


def _mesh():
    return plsc.VectorSubcoreMesh(core_axis_name="c", subcore_axis_name="s",
                                  num_cores=NC, num_subcores=NS)


def _zero_fill(zbuf):
    def body(i, _):
        for q in range(4):
            zbuf[i, pl.ds(q * 16, 16)] = jnp.zeros((16,), _f32)
        return 0
    lax.fori_loop(0, 128, body, 0)


def _deg_body(col_hbm, out_hbm, deg_sh, idxc_v, ones_v, zrow_v):
    c = lax.axis_index("c")
    s = lax.axis_index("s")
    w = c * NS + s
    pltpu.sync_copy(col_hbm.at[w], idxc_v)
    for q in range(8):
        ones_v[pl.ds(q * 16, 16)] = jnp.ones((16,), _f32)
        zrow_v[pl.ds(q * 16, 16)] = jnp.zeros((16,), _f32)
    for k in range(RPW // CH):
        pltpu.sync_copy(zrow_v, deg_sh.at[pl.ds(s * RPW + k * CH, CH)])
    plsc.subcore_barrier()

    def chunk(j, _):
        pltpu.sync_copy(ones_v, deg_sh.at[idxc_v.at[j]], add=True)
        return 0
    lax.fori_loop(0, NCH, chunk, 0)
    plsc.subcore_barrier()
    pltpu.sync_copy(deg_sh.at[pl.ds(s * RPW, RPW)],
                    out_hbm.at[c, pl.ds(s * RPW, RPW)])


def _deg_call(col_idx):
    f = pl.kernel(
        _deg_body,
        out_type=jax.ShapeDtypeStruct((NC, NP), _f32),
        mesh=_mesh(),
        scratch_types=[
            pltpu.VMEM_SHARED((NP,), _f32),
            pltpu.VMEM((NCH, CH), jnp.int32),
            pltpu.VMEM((CH,), _f32),
            pltpu.VMEM((CH,), _f32),
        ],
    )
    return f(col_idx)


def _sprop_body(u_hbm, row_hbm, col_hbm, out_hbm,
                acc_sh, idxr_v, idxc_v, b0, b1, b2, b3,
                sg0, sg1, sg2, sg3, ss0, ss1, ss2, ss3):
    c = lax.axis_index("c")
    s = lax.axis_index("s")
    w = c * NS + s
    bufs = (b0, b1, b2, b3)
    sgs = (sg0, sg1, sg2, sg3)
    sss = (ss0, ss1, ss2, ss3)
    pltpu.sync_copy(row_hbm.at[w], idxr_v)
    pltpu.sync_copy(col_hbm.at[w], idxc_v)
    _zero_fill(b0)
    for k in range(RPW // CH):
        pltpu.sync_copy(b0, acc_sh.at[pl.ds(s * RPW + k * CH, CH)])
    plsc.subcore_barrier()

    pltpu.async_copy(u_hbm.at[idxr_v.at[0]], b0, sg0)
    pltpu.async_copy(u_hbm.at[idxr_v.at[1]], b1, sg1)

    def quad(t, _):
        for u in range(4):
            j = 4 * t + u
            v = (u + 2) % 4
            pltpu.make_async_copy(u_hbm.at[idxr_v.at[j]], bufs[u],
                                  sgs[u]).wait()
            pltpu.async_copy(bufs[u], acc_sh.at[idxc_v.at[j]], sss[u],
                             add=True)

            @pl.when(j >= 2)
            def _():
                pltpu.make_async_copy(bufs[v], acc_sh.at[idxc_v.at[j - 2]],
                                      sss[v]).wait()

            @pl.when(j + 2 < NCH)
            def _():
                pltpu.async_copy(u_hbm.at[idxr_v.at[j + 2]], bufs[v], sgs[v])
        return 0
    lax.fori_loop(0, NCH // 4, quad, 0)
    pltpu.make_async_copy(b2, acc_sh.at[idxc_v.at[NCH - 2]], ss2).wait()
    pltpu.make_async_copy(b3, acc_sh.at[idxc_v.at(NCH - 1) if False else idxc_v.at[NCH - 1]], ss3).wait()
    plsc.subcore_barrier()
    pltpu.sync_copy(acc_sh.at[pl.ds(s * RPW, RPW)],
                    out_hbm.at[c, pl.ds(s * RPW, RPW)])


def _sprop_call(u, row_idx, col_idx):
    f = pl.kernel(
        _sprop_body,
        out_type=jax.ShapeDtypeStruct((NC, NP, D), _f32),
        mesh=_mesh(),
        compiler_params=pltpu.CompilerParams(use_tc_tiling_on_sc=False),
        scratch_types=[
            pltpu.VMEM_SHARED((NP, D), _f32),
            pltpu.VMEM((NCH, CH), jnp.int32),
            pltpu.VMEM((NCH, CH), jnp.int32),
            pltpu.VMEM((CH, D), _f32),
            pltpu.VMEM((CH, D), _f32),
            pltpu.VMEM((CH, D), _f32),
            pltpu.VMEM((CH, D), _f32),
            pltpu.SemaphoreType.DMA,
            pltpu.SemaphoreType.DMA,
            pltpu.SemaphoreType.DMA,
            pltpu.SemaphoreType.DMA,
            pltpu.SemaphoreType.DMA,
            pltpu.SemaphoreType.DMA,
            pltpu.SemaphoreType.DMA,
            pltpu.SemaphoreType.DMA,
        ],
    )
    return f(u, row_idx, col_idx)


def _gat_body(g_hbm, asrc_hbm, adst_hbm, c_hbm, row_hbm, col_hbm,
              acc_out, den_out,
              acc_sh, den_sh,
              idxr_v, idxc_v, g0, g1, g2, g3, av0, av1, bv0, bv1,
              ee0, ee1, c_v,
              sg0, sg1, sg2, sg3, ssa0, ssa1, ssa2, ssa3,
              sa0, sa1, sb0, sb1, sd0, sd1):
    c = lax.axis_index("c")
    s = lax.axis_index("s")
    w = c * NS + s
    gb = (g0, g1, g2, g3)
    sg = (sg0, sg1, sg2, sg3)
    ssa = (ssa0, ssa1, ssa2, ssa3)
    av = (av0, av1)
    bv = (bv0, bv1)
    sa = (sa0, sa1)
    sb = (sb0, sb1)
    ee = (ee0, ee1)
    sd = (sd0, sd1)
    pltpu.sync_copy(row_hbm.at[w], idxr_v)
    pltpu.sync_copy(col_hbm.at[w], idxc_v)
    pltpu.sync_copy(c_hbm.at[pl.ds(0, 16)], c_v)
    _zero_fill(g0)
    for q in range(8):
        ee0[pl.ds(q * 16, 16)] = jnp.zeros((16,), _f32)
    for k in range(RPW // CH):
        pltpu.sync_copy(g0, acc_sh.at[pl.ds(s * RPW + k * CH, CH)])
        pltpu.sync_copy(ee0, den_sh.at[pl.ds(s * RPW + k * CH, CH)])
    plsc.subcore_barrier()

    cv = c_v[...]

    for b in range(2):
        pltpu.async_copy(asrc_hbm.at[idxr_v.at[b]], av[b], sa[b])
        pltpu.async_copy(adst_hbm.at[idxc_v.at[b]], bv[b], sb[b])
        pltpu.async_copy(g_hbm.at[idxr_v.at[b]], gb[b], sg[b])

    def quad(t, _):
        for u in range(4):
            j = 4 * t + u
            p = u % 2
            v = (u + 2) % 4
            pltpu.make_async_copy(asrc_hbm.at[idxr_v.at[j]], av[p],
                                  sa[p]).wait()
            pltpu.make_async_copy(adst_hbm.at[idxc_v.at[j]], bv[p],
                                  sb[p]).wait()

            @pl.when(j >= 2)
            def _():
                pltpu.make_async_copy(ee[p], den_sh.at[idxc_v.at[j - 2]],
                                      sd[p]).wait()
            for i in range(CH // 16):
                sv = av[p][pl.ds(i * 16, 16)] + bv[p][pl.ds(i * 16, 16)]
                ev = jnp.where(sv > 0, sv, 0.2 * sv)
                ee[p][pl.ds(i * 16, 16)] = jnp.exp(ev - cv)
            pltpu.async_copy(ee[p], den_sh.at[idxc_v.at[j]], sd[p], add=True)

            @pl.when(j + 2 < NCH)
            def _():
                pltpu.async_copy(asrc_hbm.at[idxr_v.at[j + 2]], av[p], sa[p])
                pltpu.async_copy(adst_hbm.at[idxc_v.at[j + 2]], bv[p], sb[p])
            pltpu.make_async_copy(g_hbm.at[idxr_v.at[j]], gb[u], sg[u]).wait()

            def scale(k, _, _u=u, _p=p):
                eev = ee[_p][pl.ds(k * 16, 16)]
                for l in range(16):
                    sc = jnp.full((16,), eev[l], _f32)
                    r = k * 16 + l
                    for q in range(4):
                        gb[_u][r, pl.ds(q * 16, 16)] = (
                            gb[_u][r, pl.ds(q * 16, 16)] * sc)
                return 0
            lax.fori_loop(0, CH // 16, scale, 0)
            pltpu.async_copy(gb[u], acc_sh.at[idxc_v.at[j]], ssa[u], add=True)

            @pl.when(j >= 2)
            def _():
                pltpu.make_async_copy(gb[v], acc_sh.at[idxc_v.at[j - 2]],
                                      ssa[v]).wait()

            @pl.when(j + 2 < NCH)
            def _():
                pltpu.async_copy(g_hbm.at[idxr_v.at[j + 2]], gb[v], sg[v])
        return 0
    lax.fori_loop(0, NCH // 4, quad, 0)
    pltpu.make_async_copy(ee0, den_sh.at[idxc_v.at[NCH - 2]], sd0).wait()
    pltpu.make_async_copy(ee1, den_sh.at[idxc_v.at[NCH - 1]], sd1).wait()
    pltpu.make_async_copy(g2, acc_sh.at[idxc_v.at[NCH - 2]], ssa2).wait()
    pltpu.make_async_copy(g3, acc_sh.at[idxc_v.at[NCH - 1]], ssa3).wait()
    plsc.subcore_barrier()
    pltpu.sync_copy(acc_sh.at[pl.ds(s * RPW, RPW)],
                    acc_out.at[c, pl.ds(s * RPW, RPW)])
    pltpu.sync_copy(den_sh.at[pl.ds(s * RPW, RPW)],
                    den_out.at[c, pl.ds(s * RPW, RPW)])


def _gat_call(g, asrc, adst, c128, row_idx, col_idx):
    f = pl.kernel(
        _gat_body,
        out_type=(jax.ShapeDtypeStruct((NC, NP, D), _f32),
                  jax.ShapeDtypeStruct((NC, NP), _f32)),
        mesh=_mesh(),
        compiler_params=pltpu.CompilerParams(use_tc_tiling_on_sc=False),
        scratch_types=[
            pltpu.VMEM_SHARED((NP, D), _f32),
            pltpu.VMEM_SHARED((NP,), _f32),
            pltpu.VMEM((NCH, CH), jnp.int32),
            pltpu.VMEM((NCH, CH), jnp.int32),
            pltpu.VMEM((CH, D), _f32),
            pltpu.VMEM((CH, D), _f32),
            pltpu.VMEM((CH, D), _f32),
            pltpu.VMEM((CH, D), _f32),
            pltpu.VMEM((CH,), _f32),
            pltpu.VMEM((CH,), _f32),
            pltpu.VMEM((CH,), _f32),
            pltpu.VMEM((CH,), _f32),
            pltpu.VMEM((CH,), _f32),
            pltpu.VMEM((CH,), _f32),
            pltpu.VMEM((16,), _f32),
        ] + [pltpu.SemaphoreType.DMA] * 14,
    )
    return f(g, asrc, adst, c128, row_idx, col_idx)


def _mmT(a, w):
    return lax.dot_general(a, w, (((1,), (1,)), ((), ())),
                           preferred_element_type=_f32)


def _lrelu(v, s):
    return jnp.where(v > 0, v, s * v)


def _tc_a_body(degp_ref, xp_ref, w1_ref, dinv_ref, dinv2_ref, u1_ref):
    deg = degp_ref[0, :] + degp_ref[1, :] + 1.0
    dv = lax.rsqrt(deg)
    dinv_ref[0, :] = dv
    dinv2_ref[0, :] = dv * dv
    y = _mmT(xp_ref[...], w1_ref[...])
    u1_ref[...] = dv[:, None] * y


def _tc_glue_body(sp_ref, u_ref, dinv2_ref, out_ref):
    s = sp_ref[0] + sp_ref[1] + u_ref[...]
    out_ref[...] = dinv2_ref[0, :][:, None] * s


def _tc_conv_body(sp_ref, u_ref, dinv_ref, b_ref, w_ref, h_ref, un_ref):
    dv = dinv_ref[0, :]
    p = dv[:, None] * (sp_ref[0] + sp_ref[1] + u_ref[...]) + b_ref[0, :]
    h = _lrelu(p, 0.1)
    h_ref[...] = h
    un_ref[...] = dv[:, None] * _mmT(h, w_ref[...])


def _tc_gate_body(sp_ref, u_ref, dinv_ref, b_ref, wg_ref, asv_ref, adv_ref,
                  g_ref, asrc_ref, adst_ref, c_ref):
    dv = dinv_ref[0, :]
    p = dv[:, None] * (sp_ref[0] + sp_ref[1] + u_ref[...]) + b_ref[0, :]
    h2 = _lrelu(p, 0.1)
    g = _mmT(h2, wg_ref[...])
    g_ref[...] = g
    asrc = _mmT(asv_ref[...], g)
    adst = _mmT(adv_ref[...], g)
    asrc_ref[...] = asrc
    adst_ref[...] = adst
    msk = lax.broadcasted_iota(jnp.int32, (1, NP), 1) < N
    m = (jnp.max(jnp.where(msk, asrc, -1e30))
         + jnp.max(jnp.where(msk, adst, -1e30)))
    cc = jnp.where(m > 0, m, 0.2 * m)
    c_ref[...] = jnp.full((1, 128), cc, _f32)


def _tc_fin_body(accp_ref, denp_ref, asrc_ref, adst_ref, c_ref, g_ref,
                 bg_ref, h3_ref):
    cc = c_ref[0, 0]
    sv = asrc_ref[0, :] + adst_ref[0, :]
    es = jnp.exp(_lrelu(sv, 0.2) - cc)
    num = accp_ref[0] + accp_ref[1] + es[:, None] * g_ref[...]
    den = denp_ref[0, :] + denp_ref[1, :] + es
    h3_ref[...] = _lrelu(num / (den[:, None] + 1e-16) + bg_ref[0, :], 0.1)


def _tc_mlp_body(z_ref, w1_ref, b1_ref, w2_ref, b2_ref, w3_ref, b3_ref,
                 out_ref):
    o = _lrelu(_mmT(z_ref[...], w1_ref[...]) + b1_ref[0, :], 0.1)
    o = _lrelu(_mmT(o, w2_ref[...]) + b2_ref[0, :], 0.1)
    out_ref[...] = _lrelu(_mmT(o, w3_ref[...]) + b3_ref[0, :], 0.1)


def _tc_call(body, out_shapes, *args):
    return pl.pallas_call(
        body,
        out_shape=out_shapes,
    )(*args)


def kernel(x, edge_index, W1, b1, W2, b2, Wg, att_src, att_dst, bg,
           Wl1, bl1, Wl2, bl2, Wl3, bl3):
    row = edge_index[0]
    col = edge_index[1]
    pad = EPWP - EPW
    padrow = ((jnp.arange(NW, dtype=jnp.int32) * 317) % N)[:, None]
    row_idx = jnp.concatenate(
        [row.reshape(NW, EPW),
         jnp.broadcast_to(padrow, (NW, pad)).astype(jnp.int32)],
        axis=1).reshape(NW, NCH, CH)
    padcol = (N + (jnp.arange(NW, dtype=jnp.int32) % 128))[:, None]
    col_idx = jnp.concatenate(
        [col.reshape(NW, EPW),
         jnp.broadcast_to(padcol, (NW, pad)).astype(jnp.int32)],
        axis=1).reshape(NW, NCH, CH)

    xp = jnp.pad(x, ((0, NP - N), (0, 0)))

    degp = _deg_call(col_idx)

    dinv, dinv2, u1 = _tc_call(
        _tc_a_body,
        (jax.ShapeDtypeStruct((1, NP), _f32),
         jax.ShapeDtypeStruct((1, NP), _f32),
         jax.ShapeDtypeStruct((NP, D), _f32)),
        degp, xp, W1)

    s1 = _sprop_call(u1, row_idx, col_idx)
    u2 = _tc_call(_tc_glue_body, jax.ShapeDtypeStruct((NP, D), _f32),
                  s1, u1, dinv2)
    s2 = _sprop_call(u2, row_idx, col_idx)
    h1, u3 = _tc_call(
        _tc_conv_body,
        (jax.ShapeDtypeStruct((NP, D), _f32),
         jax.ShapeDtypeStruct((NP, D), _f32)),
        s2, u2, dinv, b1.reshape(1, D), W2)
    s3 = _sprop_call(u3, row_idx, col_idx)
    u4 = _tc_call(_tc_glue_body, jax.ShapeDtypeStruct((NP, D), _f32),
                  s3, u3, dinv2)
    s4 = _sprop_call(u4, row_idx, col_idx)

    g, asrc, adst, c128 = _tc_call(
        _tc_gate_body,
        (jax.ShapeDtypeStruct((NP, D), _f32),
         jax.ShapeDtypeStruct((1, NP), _f32),
         jax.ShapeDtypeStruct((1, NP), _f32),
         jax.ShapeDtypeStruct((1, 128), _f32)),
        s4, u4, dinv, b2.reshape(1, D), Wg,
        att_src.reshape(1, D), att_dst.reshape(1, D))

    accp, denp = _gat_call(g, asrc.reshape(NP), adst.reshape(NP),
                           c128.reshape(128), row_idx, col_idx)

    h3 = _tc_call(
        _tc_fin_body, jax.ShapeDtypeStruct((NP, D), _f32),
        accp, denp, asrc, adst, c128, g, bg.reshape(1, D))

    z = h3[:N].reshape(N // 40, 40 * D)
    out = _tc_call(
        _tc_mlp_body, jax.ShapeDtypeStruct((N // 40, 10), _f32),
        z, Wl1, bl1.reshape(1, -1), Wl2, bl2.reshape(1, -1),
        Wl3, bl3.reshape(1, -1))
    return out
